# baseline (speedup 1.0000x reference)
import functools

import jax
import jax.numpy as jnp
from jax import lax
from jax.experimental import pallas as pl
from jax.experimental.pallas import tpu as pltpu
from jax.experimental.pallas import tpu_sc as plsc

VOCAB = 1000000
DIM = 64
B = 4096
L = 200
EPS = 1e-05

NC = 2
NS = 16
NW = NC * NS
BL = B * L
CHUNK = 128
ROWS_PER_W = BL // NW
NCHUNK = ROWS_PER_W // CHUNK

_GATHER_DNUMS = lax.GatherDimensionNumbers(
    offset_dims=(), collapsed_slice_dims=(0,), start_index_map=(0,))


def _lane_shuffle(v, idx):
    return lax.gather(
        v, idx[:, None], dimension_numbers=_GATHER_DNUMS, slice_sizes=(1,),
        mode=lax.GatherScatterMode.PROMISE_IN_BOUNDS)


def _rsqrt(x):
    i = jax.lax.bitcast_convert_type(x, jnp.int32)
    i = jnp.int32(0x5F3759DF) - jax.lax.shift_right_arithmetic(i, 1)
    y = jax.lax.bitcast_convert_type(i, jnp.float32)
    for _ in range(2):
        y = y * (1.5 - 0.5 * x * y * y)
    return y


def _layernorm_chunk(rows_v, out_v, g_vecs, b_vecs):
    inv_d = 1.0 / DIM
    lane = lax.iota(jnp.int32, 16)
    lo_half = lane < 8
    idx8 = lane ^ 8
    idx_a = jnp.zeros((16,), jnp.int32)
    idx_b = jnp.full((16,), 8, jnp.int32)

    @plsc.parallel_loop(0, CHUNK, step=2, unroll=8)
    def pair_body(r):
        va = [rows_v[r, pl.ds(16 * k, 16)] for k in range(4)]
        vb = [rows_v[r + 1, pl.ds(16 * k, 16)] for k in range(4)]
        sa = (va[0] + va[1]) + (va[2] + va[3])
        sb = (vb[0] + vb[1]) + (vb[2] + vb[3])
        ta = (va[0] * va[0] + va[1] * va[1]) + (va[2] * va[2] + va[3] * va[3])
        tb = (vb[0] * vb[0] + vb[1] * vb[1]) + (vb[2] * vb[2] + vb[3] * vb[3])
        sa = sa + _lane_shuffle(sa, idx8)
        sb = sb + _lane_shuffle(sb, idx8)
        ta = ta + _lane_shuffle(ta, idx8)
        tb = tb + _lane_shuffle(tb, idx8)
        s = jnp.where(lo_half, sa, sb)
        t = jnp.where(lo_half, ta, tb)
        for sh in (4, 2, 1):
            idx = lane ^ sh
            s = s + _lane_shuffle(s, idx)
            t = t + _lane_shuffle(t, idx)
        mean = s * inv_d
        var = t * inv_d - mean * mean
        rstd = _rsqrt(var + EPS)
        u = mean * rstd
        rstd_a = _lane_shuffle(rstd, idx_a)
        rstd_b = _lane_shuffle(rstd, idx_b)
        u_a = _lane_shuffle(u, idx_a)
        u_b = _lane_shuffle(u, idx_b)
        for k in range(4):
            out_v[r, pl.ds(16 * k, 16)] = (va[k] * rstd_a - u_a) * g_vecs[k] + b_vecs[k]
            out_v[r + 1, pl.ds(16 * k, 16)] = (vb[k] * rstd_b - u_b) * g_vecs[k] + b_vecs[k]


def _make_sc_kernel():
    mesh = plsc.VectorSubcoreMesh(core_axis_name="c", subcore_axis_name="s")

    @functools.partial(
        pl.kernel,
        out_type=jax.ShapeDtypeStruct((BL, DIM), jnp.float32),
        mesh=mesh,
        compiler_params=pltpu.CompilerParams(use_tc_tiling_on_sc=True),
        scratch_types=[
            pltpu.VMEM((NCHUNK, CHUNK), jnp.int32),
            pltpu.VMEM((CHUNK, 128), jnp.float32),
            pltpu.VMEM((CHUNK, 128), jnp.float32),
            pltpu.VMEM((CHUNK, DIM), jnp.float32),
            pltpu.VMEM((CHUNK, DIM), jnp.float32),
            pltpu.VMEM((DIM,), jnp.float32),
            pltpu.VMEM((DIM,), jnp.float32),
            pltpu.SemaphoreType.DMA,
            pltpu.SemaphoreType.DMA,
            pltpu.SemaphoreType.DMA,
            pltpu.SemaphoreType.DMA,
        ],
    )
    def sc_kernel(x_hbm, table_hbm, gamma_hbm, beta_hbm, out_hbm,
                  idx_v, rows0, rows1, outv0, outv1, gam_v, bet_v,
                  gsem0, gsem1, ssem0, ssem1):
        rows = (rows0, rows1)
        outv = (outv0, outv1)
        gsem = (gsem0, gsem1)
        ssem = (ssem0, ssem1)

        wid = lax.axis_index("s") * NC + lax.axis_index("c")
        chunk0 = wid * NCHUNK

        pltpu.sync_copy(gamma_hbm, gam_v)
        pltpu.sync_copy(beta_hbm, bet_v)
        g_vecs = [gam_v[pl.ds(16 * k, 16)] for k in range(4)]
        b_vecs = [bet_v[pl.ds(16 * k, 16)] for k in range(4)]

        pltpu.sync_copy(x_hbm.at[pl.ds(chunk0, NCHUNK)], idx_v)

        def fire_gather(g, b):
            pltpu.async_copy(table_hbm.at[idx_v.at[g]], rows[b], gsem[b])

        def wait_gather(b):
            pltpu.make_async_copy(
                table_hbm.at[pl.ds(0, CHUNK)], rows[b], gsem[b]).wait()

        def fire_scatter(g, b):
            dst = out_hbm.at[pl.ds((chunk0 + g) * CHUNK, CHUNK)]
            pltpu.async_copy(outv[b], dst, ssem[b])

        def wait_scatter(b):
            pltpu.make_async_copy(
                outv[b], out_hbm.at[pl.ds(0, CHUNK)], ssem[b]).wait()

        fire_gather(0, 0)
        fire_gather(1, 1)
        for b in (0, 1):
            wait_gather(b)
            _layernorm_chunk(rows[b], outv[b], g_vecs, b_vecs)
            fire_scatter(b, b)
            fire_gather(b + 2, b)

        def steady(k, _):
            for b in (0, 1):
                g = 2 * k + b
                wait_gather(b)
                wait_scatter(b)
                _layernorm_chunk(rows[b], outv[b], g_vecs, b_vecs)
                fire_scatter(g, b)
                fire_gather(g + 2, b)
            return 0

        lax.fori_loop(1, NCHUNK // 2 - 1, steady, 0)

        for b in (0, 1):
            g = NCHUNK - 2 + b
            wait_gather(b)
            wait_scatter(b)
            _layernorm_chunk(rows[b], outv[b], g_vecs, b_vecs)
            fire_scatter(g, b)
        for b in (0, 1):
            wait_scatter(b)

    return sc_kernel


_SC_KERNEL = _make_sc_kernel()


def kernel(x, table, gamma, beta):
    x2 = x.reshape(BL // CHUNK, CHUNK)
    table128 = jnp.concatenate(
        [table, jnp.zeros((VOCAB, 128 - DIM), jnp.float32)], axis=1)
    out = _SC_KERNEL(x2, table128, gamma, beta)
    return out.reshape(B, L, DIM)


# speedup vs baseline: 1.2414x; 1.2414x over previous
import functools

import jax
import jax.numpy as jnp
from jax import lax
from jax.experimental import pallas as pl
from jax.experimental.pallas import tpu as pltpu
from jax.experimental.pallas import tpu_sc as plsc

VOCAB = 1000000
DIM = 64
B = 4096
L = 200
EPS = 1e-05

NC = 2
NS = 16
NW = NC * NS
BL = B * L
CHUNK = 128
ROWS_PER_W = BL // NW
NCHUNK = ROWS_PER_W // CHUNK

_GATHER_DNUMS = lax.GatherDimensionNumbers(
    offset_dims=(), collapsed_slice_dims=(0,), start_index_map=(0,))


def _lane_shuffle(v, idx):
    return lax.gather(
        v, idx[:, None], dimension_numbers=_GATHER_DNUMS, slice_sizes=(1,),
        mode=lax.GatherScatterMode.PROMISE_IN_BOUNDS)


def _rsqrt(x):
    i = jax.lax.bitcast_convert_type(x, jnp.int32)
    i = jnp.int32(0x5F3759DF) - jax.lax.shift_right_arithmetic(i, 1)
    y = jax.lax.bitcast_convert_type(i, jnp.float32)
    for _ in range(2):
        y = y * (1.5 - 0.5 * x * y * y)
    return y


def _layernorm_chunk(rows_v, out_v, g_vecs, b_vecs):
    inv_d = 1.0 / DIM
    lane = lax.iota(jnp.int32, 16)
    lo_half = lane < 8
    idx8 = lane ^ 8
    idx_a = jnp.zeros((16,), jnp.int32)
    idx_b = jnp.full((16,), 8, jnp.int32)

    @plsc.parallel_loop(0, CHUNK, step=2, unroll=4)
    def pair_body(r):
        va = [rows_v[r, pl.ds(16 * k, 16)] for k in range(4)]
        vb = [rows_v[r + 1, pl.ds(16 * k, 16)] for k in range(4)]
        sa = (va[0] + va[1]) + (va[2] + va[3])
        sb = (vb[0] + vb[1]) + (vb[2] + vb[3])
        ta = (va[0] * va[0] + va[1] * va[1]) + (va[2] * va[2] + va[3] * va[3])
        tb = (vb[0] * vb[0] + vb[1] * vb[1]) + (vb[2] * vb[2] + vb[3] * vb[3])
        sa = sa + _lane_shuffle(sa, idx8)
        sb = sb + _lane_shuffle(sb, idx8)
        ta = ta + _lane_shuffle(ta, idx8)
        tb = tb + _lane_shuffle(tb, idx8)
        s = jnp.where(lo_half, sa, sb)
        t = jnp.where(lo_half, ta, tb)
        for sh in (4, 2, 1):
            idx = lane ^ sh
            s = s + _lane_shuffle(s, idx)
            t = t + _lane_shuffle(t, idx)
        mean = s * inv_d
        var = t * inv_d - mean * mean
        rstd = _rsqrt(var + EPS)
        u = mean * rstd
        rstd_a = _lane_shuffle(rstd, idx_a)
        rstd_b = _lane_shuffle(rstd, idx_b)
        u_a = _lane_shuffle(u, idx_a)
        u_b = _lane_shuffle(u, idx_b)
        for k in range(4):
            out_v[r, pl.ds(16 * k, 16)] = (va[k] * rstd_a - u_a) * g_vecs[k] + b_vecs[k]
            out_v[r + 1, pl.ds(16 * k, 16)] = (vb[k] * rstd_b - u_b) * g_vecs[k] + b_vecs[k]


def _make_sc_kernel():
    mesh = plsc.VectorSubcoreMesh(core_axis_name="c", subcore_axis_name="s")

    @functools.partial(
        pl.kernel,
        out_type=jax.ShapeDtypeStruct((BL, DIM), jnp.float32),
        mesh=mesh,
        compiler_params=pltpu.CompilerParams(use_tc_tiling_on_sc=True),
        scratch_types=[
            pltpu.VMEM((NCHUNK, CHUNK), jnp.int32),
            pltpu.VMEM((CHUNK, 128), jnp.float32),
            pltpu.VMEM((CHUNK, 128), jnp.float32),
            pltpu.VMEM((CHUNK, DIM), jnp.float32),
            pltpu.VMEM((CHUNK, DIM), jnp.float32),
            pltpu.VMEM((DIM,), jnp.float32),
            pltpu.VMEM((DIM,), jnp.float32),
            pltpu.SemaphoreType.DMA,
            pltpu.SemaphoreType.DMA,
            pltpu.SemaphoreType.DMA,
            pltpu.SemaphoreType.DMA,
        ],
    )
    def sc_kernel(x_hbm, table_hbm, gamma_hbm, beta_hbm, out_hbm,
                  idx_v, rows0, rows1, outv0, outv1, gam_v, bet_v,
                  gsem0, gsem1, ssem0, ssem1):
        rows = (rows0, rows1)
        outv = (outv0, outv1)
        gsem = (gsem0, gsem1)
        ssem = (ssem0, ssem1)

        wid = lax.axis_index("s") * NC + lax.axis_index("c")
        chunk0 = wid * NCHUNK

        pltpu.sync_copy(gamma_hbm, gam_v)
        pltpu.sync_copy(beta_hbm, bet_v)
        g_vecs = [gam_v[pl.ds(16 * k, 16)] for k in range(4)]
        b_vecs = [bet_v[pl.ds(16 * k, 16)] for k in range(4)]

        pltpu.sync_copy(x_hbm.at[pl.ds(chunk0, NCHUNK)], idx_v)

        def fire_gather(g, b):
            pltpu.async_copy(table_hbm.at[idx_v.at[g]], rows[b], gsem[b])

        def wait_gather(b):
            pltpu.make_async_copy(
                table_hbm.at[pl.ds(0, CHUNK)], rows[b], gsem[b]).wait()

        def fire_scatter(g, b):
            dst = out_hbm.at[pl.ds((chunk0 + g) * CHUNK, CHUNK)]
            pltpu.async_copy(outv[b], dst, ssem[b])

        def wait_scatter(b):
            pltpu.make_async_copy(
                outv[b], out_hbm.at[pl.ds(0, CHUNK)], ssem[b]).wait()

        fire_gather(0, 0)
        fire_gather(1, 1)
        for b in (0, 1):
            wait_gather(b)
            _layernorm_chunk(rows[b], outv[b], g_vecs, b_vecs)
            fire_scatter(b, b)
            fire_gather(b + 2, b)

        def steady(k, _):
            for b in (0, 1):
                g = 2 * k + b
                wait_gather(b)
                wait_scatter(b)
                _layernorm_chunk(rows[b], outv[b], g_vecs, b_vecs)
                fire_scatter(g, b)
                fire_gather(g + 2, b)
            return 0

        lax.fori_loop(1, NCHUNK // 2 - 1, steady, 0)

        for b in (0, 1):
            g = NCHUNK - 2 + b
            wait_gather(b)
            wait_scatter(b)
            _layernorm_chunk(rows[b], outv[b], g_vecs, b_vecs)
            fire_scatter(g, b)
        for b in (0, 1):
            wait_scatter(b)

    return sc_kernel


_SC_KERNEL = _make_sc_kernel()


def kernel(x, table, gamma, beta):
    x2 = x.reshape(BL // CHUNK, CHUNK)
    table128 = jnp.concatenate(
        [table, jnp.zeros((VOCAB, 128 - DIM), jnp.float32)], axis=1)
    out = _SC_KERNEL(x2, table128, gamma, beta)
    return out.reshape(B, L, DIM)


# speedup vs baseline: 1.2701x; 1.0231x over previous
import functools

import jax
import jax.numpy as jnp
from jax import lax
from jax.experimental import pallas as pl
from jax.experimental.pallas import tpu as pltpu
from jax.experimental.pallas import tpu_sc as plsc

VOCAB = 1000000
DIM = 64
B = 4096
L = 200
EPS = 1e-05

NC = 2
NS = 16
NW = NC * NS
BL = B * L
CHUNK = 128
ROWS_PER_W = BL // NW
NCHUNK = ROWS_PER_W // CHUNK

_GATHER_DNUMS = lax.GatherDimensionNumbers(
    offset_dims=(), collapsed_slice_dims=(0,), start_index_map=(0,))


def _lane_shuffle(v, idx):
    return lax.gather(
        v, idx[:, None], dimension_numbers=_GATHER_DNUMS, slice_sizes=(1,),
        mode=lax.GatherScatterMode.PROMISE_IN_BOUNDS)


def _rsqrt(x):
    i = jax.lax.bitcast_convert_type(x, jnp.int32)
    i = jnp.int32(0x5F3759DF) - jax.lax.shift_right_arithmetic(i, 1)
    y = jax.lax.bitcast_convert_type(i, jnp.float32)
    for _ in range(2):
        y = y * (1.5 - 0.5 * x * y * y)
    return y


def _layernorm_chunk(rows_v, out_v, g_vecs, b_vecs):
    inv_d = 1.0 / DIM
    lane = lax.iota(jnp.int32, 16)
    lo_half = lane < 8
    idx8 = lane ^ 8
    idx_a = jnp.zeros((16,), jnp.int32)
    idx_b = jnp.full((16,), 8, jnp.int32)

    @plsc.parallel_loop(0, CHUNK, step=2, unroll=4)
    def pair_body(r):
        va = [rows_v[r, pl.ds(16 * k, 16)] for k in range(4)]
        vb = [rows_v[r + 1, pl.ds(16 * k, 16)] for k in range(4)]
        sa = (va[0] + va[1]) + (va[2] + va[3])
        sb = (vb[0] + vb[1]) + (vb[2] + vb[3])
        ta = (va[0] * va[0] + va[1] * va[1]) + (va[2] * va[2] + va[3] * va[3])
        tb = (vb[0] * vb[0] + vb[1] * vb[1]) + (vb[2] * vb[2] + vb[3] * vb[3])
        sa = sa + _lane_shuffle(sa, idx8)
        sb = sb + _lane_shuffle(sb, idx8)
        ta = ta + _lane_shuffle(ta, idx8)
        tb = tb + _lane_shuffle(tb, idx8)
        s = jnp.where(lo_half, sa, sb)
        t = jnp.where(lo_half, ta, tb)
        for sh in (4, 2, 1):
            idx = lane ^ sh
            s = s + _lane_shuffle(s, idx)
            t = t + _lane_shuffle(t, idx)
        mean = s * inv_d
        var = t * inv_d - mean * mean
        rstd = _rsqrt(var + EPS)
        u = mean * rstd
        rstd_a = _lane_shuffle(rstd, idx_a)
        rstd_b = _lane_shuffle(rstd, idx_b)
        u_a = _lane_shuffle(u, idx_a)
        u_b = _lane_shuffle(u, idx_b)
        for k in range(4):
            out_v[r, pl.ds(16 * k, 16)] = (va[k] * rstd_a - u_a) * g_vecs[k] + b_vecs[k]
            out_v[r + 1, pl.ds(16 * k, 16)] = (vb[k] * rstd_b - u_b) * g_vecs[k] + b_vecs[k]


def _make_sc_kernel():
    mesh = plsc.VectorSubcoreMesh(core_axis_name="c", subcore_axis_name="s")

    @functools.partial(
        pl.kernel,
        out_type=jax.ShapeDtypeStruct((BL, DIM), jnp.float32),
        mesh=mesh,
        compiler_params=pltpu.CompilerParams(use_tc_tiling_on_sc=True),
        scratch_types=[
            pltpu.VMEM((NCHUNK, CHUNK), jnp.int32),
            pltpu.VMEM((CHUNK, 128), jnp.float32),
            pltpu.VMEM((CHUNK, 128), jnp.float32),
            pltpu.VMEM((CHUNK, 128), jnp.float32),
            pltpu.VMEM((CHUNK, DIM), jnp.float32),
            pltpu.VMEM((CHUNK, DIM), jnp.float32),
            pltpu.VMEM((CHUNK, DIM), jnp.float32),
            pltpu.VMEM((DIM,), jnp.float32),
            pltpu.VMEM((DIM,), jnp.float32),
            pltpu.SemaphoreType.DMA,
            pltpu.SemaphoreType.DMA,
            pltpu.SemaphoreType.DMA,
            pltpu.SemaphoreType.DMA,
            pltpu.SemaphoreType.DMA,
            pltpu.SemaphoreType.DMA,
        ],
    )
    def sc_kernel(x_hbm, table_hbm, gamma_hbm, beta_hbm, out_hbm,
                  idx_v, rows0, rows1, rows2, outv0, outv1, outv2,
                  gam_v, bet_v, gsem0, gsem1, gsem2, ssem0, ssem1, ssem2):
        rows = (rows0, rows1, rows2)
        outv = (outv0, outv1, outv2)
        gsem = (gsem0, gsem1, gsem2)
        ssem = (ssem0, ssem1, ssem2)

        wid = lax.axis_index("s") * NC + lax.axis_index("c")
        chunk0 = wid * NCHUNK

        pltpu.sync_copy(gamma_hbm, gam_v)
        pltpu.sync_copy(beta_hbm, bet_v)
        g_vecs = [gam_v[pl.ds(16 * k, 16)] for k in range(4)]
        b_vecs = [bet_v[pl.ds(16 * k, 16)] for k in range(4)]

        pltpu.sync_copy(x_hbm.at[pl.ds(chunk0, NCHUNK)], idx_v)

        def fire_gather(g, b):
            pltpu.async_copy(table_hbm.at[idx_v.at[g]], rows[b], gsem[b])

        def wait_gather(b):
            pltpu.make_async_copy(
                table_hbm.at[pl.ds(0, CHUNK)], rows[b], gsem[b]).wait()

        def fire_scatter(g, b):
            dst = out_hbm.at[pl.ds((chunk0 + g) * CHUNK, CHUNK)]
            pltpu.async_copy(outv[b], dst, ssem[b])

        def wait_scatter(b):
            pltpu.make_async_copy(
                outv[b], out_hbm.at[pl.ds(0, CHUNK)], ssem[b]).wait()

        for b in (0, 1, 2):
            fire_gather(b, b)
        for b in (0, 1, 2):
            wait_gather(b)
            _layernorm_chunk(rows[b], outv[b], g_vecs, b_vecs)
            fire_scatter(b, b)
            fire_gather(b + 3, b)

        def steady(k, _):
            for b in (0, 1, 2):
                g = 3 * k + b
                wait_gather(b)
                wait_scatter(b)
                _layernorm_chunk(rows[b], outv[b], g_vecs, b_vecs)
                fire_scatter(g, b)
                fire_gather(g + 3, b)
            return 0

        lax.fori_loop(1, (NCHUNK - 5) // 3, steady, 0)

        for g in range(NCHUNK - 5, NCHUNK):
            b = g % 3
            wait_gather(b)
            wait_scatter(b)
            _layernorm_chunk(rows[b], outv[b], g_vecs, b_vecs)
            fire_scatter(g, b)
            if g + 3 < NCHUNK:
                fire_gather(g + 3, b)
        for b in (0, 1, 2):
            wait_scatter(b)

    return sc_kernel


_SC_KERNEL = _make_sc_kernel()


def kernel(x, table, gamma, beta):
    x2 = x.reshape(BL // CHUNK, CHUNK)
    table128 = jnp.concatenate(
        [table, jnp.zeros((VOCAB, 128 - DIM), jnp.float32)], axis=1)
    out = _SC_KERNEL(x2, table128, gamma, beta)
    return out.reshape(B, L, DIM)
